# baseline (device time: 88207 ns/iter reference)
import jax
import jax.numpy as jnp
from jax import lax
from jax.experimental import pallas as pl
from jax.experimental.pallas import tpu as pltpu

N_DEV = 4
M = 2048
CHUNK = M // N_DEV


def kernel(x, w_mat):
    k_per, n = w_mat.shape
    m, _ = x.shape

    def body(x_ref, w_ref, out_ref, comm_ref, send_sems, recv_sems):
        my = lax.axis_index("i")
        left = (my + N_DEV - 1) % N_DEV
        right = (my + 1) % N_DEV

        barrier_sem = pltpu.get_barrier_semaphore()
        for nbr in (left, right):
            pl.semaphore_signal(
                barrier_sem, inc=1,
                device_id=(nbr,), device_id_type=pl.DeviceIdType.MESH,
            )
        pl.semaphore_wait(barrier_sem, 2)

        def partial_chunk(c):
            return jnp.dot(
                x_ref[pl.ds(c * CHUNK, CHUNK), :],
                w_ref[:, :],
                preferred_element_type=jnp.float32,
            )

        c0 = (my + N_DEV - 1) % N_DEV
        comm_ref[0] = partial_chunk(c0).astype(jnp.bfloat16)

        for h in range(N_DEV - 1):
            rdma = pltpu.make_async_remote_copy(
                src_ref=comm_ref.at[h],
                dst_ref=comm_ref.at[h + 1],
                send_sem=send_sems.at[h],
                recv_sem=recv_sems.at[h],
                device_id=(right,),
                device_id_type=pl.DeviceIdType.MESH,
            )
            rdma.start()
            c_recv = (my + (N_DEV - h - 2)) % N_DEV
            partial = partial_chunk(c_recv)
            rdma.wait()
            acc = partial + comm_ref[h + 1].astype(jnp.float32)
            if h < N_DEV - 2:
                comm_ref[h + 1] = acc.astype(jnp.bfloat16)
            else:
                out_ref[:, :] = acc * jax.nn.sigmoid(acc)

    return pl.pallas_call(
        body,
        out_shape=jax.ShapeDtypeStruct((CHUNK, n), jnp.float32),
        in_specs=[
            pl.BlockSpec(memory_space=pltpu.VMEM),
            pl.BlockSpec(memory_space=pltpu.VMEM),
        ],
        out_specs=pl.BlockSpec(memory_space=pltpu.VMEM),
        scratch_shapes=[
            pltpu.VMEM((N_DEV, CHUNK, n), jnp.bfloat16),
            pltpu.SemaphoreType.DMA((N_DEV - 1,)),
            pltpu.SemaphoreType.DMA((N_DEV - 1,)),
        ],
        compiler_params=pltpu.CompilerParams(collective_id=0),
    )(x, w_mat)


# device time: 55202 ns/iter; 1.5979x vs baseline; 1.5979x over previous
import jax
import jax.numpy as jnp
from jax import lax
from jax.experimental import pallas as pl
from jax.experimental.pallas import tpu as pltpu

N_DEV = 4
M = 2048
CHUNK = M // N_DEV


def kernel(x, w_mat):
    k_per, n = w_mat.shape
    half = n // 2

    def body(x_ref, w_ref, out_ref, comm_a, comm_b,
             send_a, recv_a, send_b, recv_b):
        my = lax.axis_index("i")
        left = (my + N_DEV - 1) % N_DEV
        right = (my + 1) % N_DEV

        barrier_sem = pltpu.get_barrier_semaphore()
        for nbr in (left, right):
            pl.semaphore_signal(
                barrier_sem, inc=1,
                device_id=(nbr,), device_id_type=pl.DeviceIdType.MESH,
            )
        pl.semaphore_wait(barrier_sem, 2)

        def partial_a(c):
            return jnp.dot(
                x_ref[pl.ds(c * CHUNK, CHUNK), :],
                w_ref[:, :half],
                preferred_element_type=jnp.float32,
            )

        def partial_b(c):
            return jnp.dot(
                x_ref[pl.ds(c * CHUNK, CHUNK), :],
                w_ref[:, half:],
                preferred_element_type=jnp.float32,
            )

        comm_a[0] = partial_a((my + N_DEV - 1) % N_DEV).astype(jnp.bfloat16)
        comm_b[0] = partial_b((my + 1) % N_DEV).astype(jnp.bfloat16)

        for h in range(N_DEV - 1):
            rdma_a = pltpu.make_async_remote_copy(
                src_ref=comm_a.at[h],
                dst_ref=comm_a.at[h + 1],
                send_sem=send_a.at[h],
                recv_sem=recv_a.at[h],
                device_id=(right,),
                device_id_type=pl.DeviceIdType.MESH,
            )
            rdma_b = pltpu.make_async_remote_copy(
                src_ref=comm_b.at[h],
                dst_ref=comm_b.at[h + 1],
                send_sem=send_b.at[h],
                recv_sem=recv_b.at[h],
                device_id=(left,),
                device_id_type=pl.DeviceIdType.MESH,
            )
            rdma_a.start()
            rdma_b.start()
            c_a = (my + (N_DEV - h - 2)) % N_DEV
            c_b = (my + h + 2) % N_DEV
            p_a = partial_a(c_a)
            p_b = partial_b(c_b)
            rdma_a.wait()
            acc_a = p_a + comm_a[h + 1].astype(jnp.float32)
            if h < N_DEV - 2:
                comm_a[h + 1] = acc_a.astype(jnp.bfloat16)
            else:
                out_ref[:, :half] = acc_a * jax.nn.sigmoid(acc_a)
            rdma_b.wait()
            acc_b = p_b + comm_b[h + 1].astype(jnp.float32)
            if h < N_DEV - 2:
                comm_b[h + 1] = acc_b.astype(jnp.bfloat16)
            else:
                out_ref[:, half:] = acc_b * jax.nn.sigmoid(acc_b)

    return pl.pallas_call(
        body,
        out_shape=jax.ShapeDtypeStruct((CHUNK, n), jnp.float32),
        in_specs=[
            pl.BlockSpec(memory_space=pltpu.VMEM),
            pl.BlockSpec(memory_space=pltpu.VMEM),
        ],
        out_specs=pl.BlockSpec(memory_space=pltpu.VMEM),
        scratch_shapes=[
            pltpu.VMEM((N_DEV, CHUNK, half), jnp.bfloat16),
            pltpu.VMEM((N_DEV, CHUNK, half), jnp.bfloat16),
            pltpu.SemaphoreType.DMA((N_DEV - 1,)),
            pltpu.SemaphoreType.DMA((N_DEV - 1,)),
            pltpu.SemaphoreType.DMA((N_DEV - 1,)),
            pltpu.SemaphoreType.DMA((N_DEV - 1,)),
        ],
        compiler_params=pltpu.CompilerParams(collective_id=0),
    )(x, w_mat)


# device time: 47016 ns/iter; 1.8761x vs baseline; 1.1741x over previous
import jax
import jax.numpy as jnp
from jax import lax
from jax.experimental import pallas as pl
from jax.experimental.pallas import tpu as pltpu

N_DEV = 4
M = 2048
CHUNK = M // N_DEV
SUB = 2
SUBR = CHUNK // SUB
N_HOP = N_DEV - 1


def kernel(x, w_mat):
    k_per, n = w_mat.shape
    half = n // 2

    def body(x_ref, w_ref, out_ref, comm_a, comm_b, part_a, part_b,
             send_a, recv_a, send_b, recv_b):
        my = lax.axis_index("i")
        left = (my + N_DEV - 1) % N_DEV
        right = (my + 1) % N_DEV

        barrier_sem = pltpu.get_barrier_semaphore()
        for nbr in (left, right):
            pl.semaphore_signal(
                barrier_sem, inc=1,
                device_id=(nbr,), device_id_type=pl.DeviceIdType.MESH,
            )
        pl.semaphore_wait(barrier_sem, 2)

        def gemm_a(c):
            return jnp.dot(
                x_ref[pl.ds(c * CHUNK, CHUNK), :],
                w_ref[:, :half],
                preferred_element_type=jnp.float32,
            )

        def gemm_b(c):
            return jnp.dot(
                x_ref[pl.ds(c * CHUNK, CHUNK), :],
                w_ref[:, half:],
                preferred_element_type=jnp.float32,
            )

        def copy_a(h, s):
            return pltpu.make_async_remote_copy(
                src_ref=comm_a.at[h, pl.ds(s * SUBR, SUBR), :],
                dst_ref=comm_a.at[h + 1, pl.ds(s * SUBR, SUBR), :],
                send_sem=send_a.at[h, s],
                recv_sem=recv_a.at[h, s],
                device_id=(right,),
                device_id_type=pl.DeviceIdType.MESH,
            )

        def copy_b(h, s):
            return pltpu.make_async_remote_copy(
                src_ref=comm_b.at[h, pl.ds(s * SUBR, SUBR), :],
                dst_ref=comm_b.at[h + 1, pl.ds(s * SUBR, SUBR), :],
                send_sem=send_b.at[h, s],
                recv_sem=recv_b.at[h, s],
                device_id=(left,),
                device_id_type=pl.DeviceIdType.MESH,
            )

        comm_a[0] = gemm_a((my + N_DEV - 1) % N_DEV).astype(jnp.bfloat16)
        for s in range(SUB):
            copy_a(0, s).start()
        comm_b[0] = gemm_b((my + 1) % N_DEV).astype(jnp.bfloat16)
        for s in range(SUB):
            copy_b(0, s).start()

        for h in range(N_HOP):
            part_a[h] = gemm_a((my + (N_DEV - h - 2)) % N_DEV)
            part_b[h] = gemm_b((my + h + 2) % N_DEV)

        for h in range(N_HOP):
            for s in range(SUB):
                rows = pl.ds(s * SUBR, SUBR)
                copy_a(h, s).wait_recv()
                acc_a = part_a[h, rows, :] + comm_a[h + 1, rows, :].astype(
                    jnp.float32)
                if h < N_HOP - 1:
                    comm_a[h + 1, rows, :] = acc_a.astype(jnp.bfloat16)
                    copy_a(h + 1, s).start()
                else:
                    out_ref[rows, :half] = acc_a * jax.nn.sigmoid(acc_a)
                copy_b(h, s).wait_recv()
                acc_b = part_b[h, rows, :] + comm_b[h + 1, rows, :].astype(
                    jnp.float32)
                if h < N_HOP - 1:
                    comm_b[h + 1, rows, :] = acc_b.astype(jnp.bfloat16)
                    copy_b(h + 1, s).start()
                else:
                    out_ref[rows, half:] = acc_b * jax.nn.sigmoid(acc_b)

        for h in range(N_HOP):
            for s in range(SUB):
                copy_a(h, s).wait_send()
                copy_b(h, s).wait_send()

    return pl.pallas_call(
        body,
        out_shape=jax.ShapeDtypeStruct((CHUNK, n), jnp.float32),
        in_specs=[
            pl.BlockSpec(memory_space=pltpu.VMEM),
            pl.BlockSpec(memory_space=pltpu.VMEM),
        ],
        out_specs=pl.BlockSpec(memory_space=pltpu.VMEM),
        scratch_shapes=[
            pltpu.VMEM((N_DEV, CHUNK, half), jnp.bfloat16),
            pltpu.VMEM((N_DEV, CHUNK, half), jnp.bfloat16),
            pltpu.VMEM((N_HOP, CHUNK, half), jnp.float32),
            pltpu.VMEM((N_HOP, CHUNK, half), jnp.float32),
            pltpu.SemaphoreType.DMA((N_HOP, SUB)),
            pltpu.SemaphoreType.DMA((N_HOP, SUB)),
            pltpu.SemaphoreType.DMA((N_HOP, SUB)),
            pltpu.SemaphoreType.DMA((N_HOP, SUB)),
        ],
        compiler_params=pltpu.CompilerParams(collective_id=0),
    )(x, w_mat)
